# baseline (speedup 1.0000x reference)
import functools

import jax
import jax.numpy as jnp
from jax import lax
from jax.experimental import pallas as pl
from jax.experimental.pallas import tpu as pltpu
from jax.experimental.pallas import tpu_sc as plsc

N = 4096
L = 16384
IN1 = 512
IN2 = 256
OUT = 256

TQ = 512


def _bn_relu_body(x_ref, w_ref, b_ref, g_ref, be_ref, o_ref):
    x = jnp.dot(x_ref[...], w_ref[...], preferred_element_type=jnp.float32)
    x = x + b_ref[...]
    m = jnp.mean(x, axis=0, keepdims=True)
    v = jnp.mean((x - m) ** 2, axis=0, keepdims=True)
    o_ref[...] = jax.nn.relu((x - m) / jnp.sqrt(v + 1e-5) * g_ref[...] + be_ref[...])


def _branch(feats, w, b, g, be):
    rows, _ = feats.shape
    return pl.pallas_call(
        _bn_relu_body,
        out_shape=jax.ShapeDtypeStruct((rows, OUT), jnp.float32),
    )(feats, w, b.reshape(1, OUT), g.reshape(1, OUT), be.reshape(1, OUT))


def _search_body(pos2_ref, b2_ref, pos1t_ref, b1_ref, idx_out_ref, wn_out_ref):
    rowmask = jax.lax.broadcasted_iota(jnp.int32, (8, N), 0) < 3
    pxt = jnp.where(rowmask, pos1t_ref[...] + b1_ref[...] * 1000.0, 0.0)
    colmask = jax.lax.broadcasted_iota(jnp.int32, (TQ, 8), 1) < 3
    pyp = jnp.where(colmask, pos2_ref[...] + b2_ref[...] * 1000.0, 0.0)

    s2 = (pxt[0:1] * pxt[0:1] + pxt[2:3] * pxt[2:3]) + pxt[1:2] * pxt[1:2]
    s1 = ((pyp[:, 0:1] * pyp[:, 0:1] + pyp[:, 2:3] * pyp[:, 2:3])
          + pyp[:, 1:2] * pyp[:, 1:2])
    dot = jax.lax.dot_general(pyp, pxt, (((1,), (0,)), ((), ())),
                              preferred_element_type=jnp.float32)
    d2 = (s1 + s2) - 2.0 * dot

    iota = jax.lax.broadcasted_iota(jnp.int32, (TQ, N), 1)
    inf = jnp.float32(jnp.inf)

    m1 = jnp.min(d2, axis=1, keepdims=True)
    i1 = jnp.min(jnp.where(d2 == m1, iota, N), axis=1, keepdims=True)
    w2 = jnp.where(iota == i1, inf, d2)
    m2 = jnp.min(w2, axis=1, keepdims=True)
    i2 = jnp.min(jnp.where(w2 == m2, iota, N), axis=1, keepdims=True)
    w3 = jnp.where(iota == i2, inf, w2)
    m3 = jnp.min(w3, axis=1, keepdims=True)
    i3 = jnp.min(jnp.where(w3 == m3, iota, N), axis=1, keepdims=True)

    wa = 1.0 / jnp.maximum(jnp.maximum(m1, 0.0), 1e-16)
    wb = 1.0 / jnp.maximum(jnp.maximum(m2, 0.0), 1e-16)
    wc = 1.0 / jnp.maximum(jnp.maximum(m3, 0.0), 1e-16)
    wsum = (wa + wb) + wc

    idx_out_ref[...] = jnp.concatenate([i1, i2, i3], axis=1)
    wn_out_ref[...] = jnp.concatenate([wa, wb, wc], axis=1) / wsum


_NW = 32
_NSPLIT = 4
_LH = L // _NSPLIT
_QW = _LH // _NW
_CQ = 32
_NCH = _QW // _CQ


def _interp_sc_kernel(f1_hbm, idx_hbm, wn_hbm, f2_hbm, out_hbm,
                      idx_a, idx_b, wn_a, wn_b, f2_a, f2_b,
                      rows_a, rows_b, out_a, out_b,
                      sem_in_a, sem_in_b, sem_g_a, sem_g_b,
                      sem_out_a, sem_out_b):
    wid = lax.axis_index("s") * 2 + lax.axis_index("c")
    idx_v = (idx_a, idx_b)
    wn_v = (wn_a, wn_b)
    f2_v = (f2_a, f2_b)
    rows_v = (rows_a, rows_b)
    out_v = (out_a, out_b)
    sem_in = (sem_in_a, sem_in_b)
    sem_g = (sem_g_a, sem_g_b)
    sem_out = (sem_out_a, sem_out_b)

    def base(c):
        return wid * _QW + c * _CQ

    def fire_in(c):
        b = c & 1
        pltpu.async_copy(idx_hbm.at[pl.ds(base(c) * 3, 3 * _CQ)], idx_v[b], sem_in[b])
        pltpu.async_copy(wn_hbm.at[pl.ds(base(c) * 3, 3 * _CQ)], wn_v[b], sem_in[b])
        pltpu.async_copy(f2_hbm.at[pl.ds(base(c), _CQ)], f2_v[b], sem_in[b])

    def wait_in(c):
        b = c & 1
        pltpu.make_async_copy(idx_hbm.at[pl.ds(base(c) * 3, 3 * _CQ)], idx_v[b], sem_in[b]).wait()
        pltpu.make_async_copy(wn_hbm.at[pl.ds(base(c) * 3, 3 * _CQ)], wn_v[b], sem_in[b]).wait()
        pltpu.make_async_copy(f2_hbm.at[pl.ds(base(c), _CQ)], f2_v[b], sem_in[b]).wait()

    def fire_g(c):
        b = c & 1
        pltpu.async_copy(f1_hbm.at[idx_v[b]], rows_v[b], sem_g[b])

    def wait_g(c):
        b = c & 1
        pltpu.make_async_copy(f1_hbm.at[idx_v[b]], rows_v[b], sem_g[b]).wait()

    def fire_out(c):
        b = c & 1
        pltpu.async_copy(out_v[b], out_hbm.at[pl.ds(base(c), _CQ)], sem_out[b])

    def wait_out(c):
        b = c & 1
        pltpu.make_async_copy(out_v[b], out_hbm.at[pl.ds(base(c), _CQ)], sem_out[b]).wait()

    fire_in(0)
    wait_in(0)
    fire_g(0)
    if _NCH > 1:
        fire_in(1)

    for c in range(_NCH):
        b = c & 1
        if c + 1 < _NCH:
            wait_in(c + 1)
            fire_g(c + 1)
        if c >= 2:
            wait_out(c - 2)
        wait_g(c)
        rows, wns, f2s, outs = rows_v[b], wn_v[b], f2_v[b], out_v[b]

        def qstep(q, _, rows=rows, wns=wns, f2s=f2s, outs=outs):
            w0 = wns[3 * q, :]
            w1 = wns[3 * q + 1, :]
            w2 = wns[3 * q + 2, :]
            for cc in range(OUT // 16):
                sl = pl.ds(cc * 16, 16)
                acc = f2s[q, sl] + w0 * rows[3 * q, sl]
                acc = acc + w1 * rows[3 * q + 1, sl]
                acc = acc + w2 * rows[3 * q + 2, sl]
                outs[q, sl] = acc
            return 0

        lax.fori_loop(0, _CQ, qstep, 0)
        fire_out(c)
        if c + 2 < _NCH:
            fire_in(c + 2)

    wait_out(_NCH - 2)
    wait_out(_NCH - 1)


def _interp_sc(f1, idx_flat, wn_flat, f2):
    return pl.kernel(
        _interp_sc_kernel,
        out_type=jax.ShapeDtypeStruct((_LH, OUT), jnp.float32),
        mesh=plsc.VectorSubcoreMesh(core_axis_name="c", subcore_axis_name="s"),
        scratch_types=[
            pltpu.VMEM((3 * _CQ,), jnp.int32),
            pltpu.VMEM((3 * _CQ,), jnp.int32),
            pltpu.VMEM((3 * _CQ, 16), jnp.float32),
            pltpu.VMEM((3 * _CQ, 16), jnp.float32),
            pltpu.VMEM((_CQ, OUT), jnp.float32),
            pltpu.VMEM((_CQ, OUT), jnp.float32),
            pltpu.VMEM((3 * _CQ, OUT), jnp.float32),
            pltpu.VMEM((3 * _CQ, OUT), jnp.float32),
            pltpu.VMEM((_CQ, OUT), jnp.float32),
            pltpu.VMEM((_CQ, OUT), jnp.float32),
            pltpu.SemaphoreType.DMA,
            pltpu.SemaphoreType.DMA,
            pltpu.SemaphoreType.DMA,
            pltpu.SemaphoreType.DMA,
            pltpu.SemaphoreType.DMA,
            pltpu.SemaphoreType.DMA,
        ],
    )(f1, idx_flat, wn_flat, f2)


def kernel(features_1, positions_1, batch_1, features_2, positions_2, batch_2,
           W1, b1, g1, be1, W2, b2, g2, be2):
    f1 = _branch(features_1, W1, b1, g1, be1)
    f2 = _branch(features_2, W2, b2, g2, be2)

    pos1t = jnp.zeros((8, N), jnp.float32).at[:3].set(positions_1.T)
    b1f = batch_1.astype(jnp.float32).reshape(1, N)
    pos2p = jnp.zeros((L, 8), jnp.float32).at[:, :3].set(positions_2)
    b2f = batch_2.astype(jnp.float32).reshape(L, 1)

    def search(pos2_half, b2_half):
        return pl.pallas_call(
            _search_body,
            grid=(_LH // TQ,),
            in_specs=[
                pl.BlockSpec((TQ, 8), lambda i: (i, 0)),
                pl.BlockSpec((TQ, 1), lambda i: (i, 0)),
                pl.BlockSpec((8, N), lambda i: (0, 0)),
                pl.BlockSpec((1, N), lambda i: (0, 0)),
            ],
            out_specs=[
                pl.BlockSpec((TQ, 3), lambda i: (i, 0)),
                pl.BlockSpec((TQ, 3), lambda i: (i, 0)),
            ],
            out_shape=[
                jax.ShapeDtypeStruct((_LH, 3), jnp.int32),
                jax.ShapeDtypeStruct((_LH, 3), jnp.float32),
            ],
        )(pos2_half, b2_half, pos1t, b1f)

    halves = []
    for h in range(_NSPLIT):
        sl = slice(h * _LH, (h + 1) * _LH)
        idx, wn = search(pos2p[sl], b2f[sl])
        wsplat = jnp.broadcast_to(wn.reshape(3 * _LH)[:, None], (3 * _LH, 16))
        halves.append(_interp_sc(f1, idx.reshape(3 * _LH), wsplat, f2[sl]))

    out = jnp.concatenate(halves, axis=0)
    return (out, positions_2, batch_2)


# speedup vs baseline: 1.2248x; 1.2248x over previous
import functools

import jax
import jax.numpy as jnp
from jax import lax
from jax.experimental import pallas as pl
from jax.experimental.pallas import tpu as pltpu
from jax.experimental.pallas import tpu_sc as plsc

N = 4096
L = 16384
IN1 = 512
IN2 = 256
OUT = 256

TQ = 512


def _bn_relu_body(x_ref, w_ref, b_ref, g_ref, be_ref, o_ref):
    x = jnp.dot(x_ref[...], w_ref[...], preferred_element_type=jnp.float32)
    x = x + b_ref[...]
    m = jnp.mean(x, axis=0, keepdims=True)
    v = jnp.mean((x - m) ** 2, axis=0, keepdims=True)
    o_ref[...] = jax.nn.relu((x - m) / jnp.sqrt(v + 1e-5) * g_ref[...] + be_ref[...])


def _branch(feats, w, b, g, be):
    rows, _ = feats.shape
    return pl.pallas_call(
        _bn_relu_body,
        out_shape=jax.ShapeDtypeStruct((rows, OUT), jnp.float32),
    )(feats, w, b.reshape(1, OUT), g.reshape(1, OUT), be.reshape(1, OUT))


def _search_body(pos2_ref, b2_ref, pos1t_ref, b1_ref, idx_out_ref, wn_out_ref):
    rowmask = jax.lax.broadcasted_iota(jnp.int32, (8, N), 0) < 3
    pxt = jnp.where(rowmask, pos1t_ref[...] + b1_ref[...] * 1000.0, 0.0)
    colmask = jax.lax.broadcasted_iota(jnp.int32, (TQ, 8), 1) < 3
    pyp = jnp.where(colmask, pos2_ref[...] + b2_ref[...] * 1000.0, 0.0)

    s2 = (pxt[0:1] * pxt[0:1] + pxt[2:3] * pxt[2:3]) + pxt[1:2] * pxt[1:2]
    s1 = ((pyp[:, 0:1] * pyp[:, 0:1] + pyp[:, 2:3] * pyp[:, 2:3])
          + pyp[:, 1:2] * pyp[:, 1:2])
    dot = jax.lax.dot_general(pyp, pxt, (((1,), (0,)), ((), ())),
                              preferred_element_type=jnp.float32)
    d2 = (s1 + s2) - 2.0 * dot

    iota = jax.lax.broadcasted_iota(jnp.int32, (TQ, N), 1)
    inf = jnp.float32(jnp.inf)

    m1 = jnp.min(d2, axis=1, keepdims=True)
    i1 = jnp.min(jnp.where(d2 == m1, iota, N), axis=1, keepdims=True)
    w2 = jnp.where(iota == i1, inf, d2)
    m2 = jnp.min(w2, axis=1, keepdims=True)
    i2 = jnp.min(jnp.where(w2 == m2, iota, N), axis=1, keepdims=True)
    w3 = jnp.where(iota == i2, inf, w2)
    m3 = jnp.min(w3, axis=1, keepdims=True)
    i3 = jnp.min(jnp.where(w3 == m3, iota, N), axis=1, keepdims=True)

    wa = 1.0 / jnp.maximum(jnp.maximum(m1, 0.0), 1e-16)
    wb = 1.0 / jnp.maximum(jnp.maximum(m2, 0.0), 1e-16)
    wc = 1.0 / jnp.maximum(jnp.maximum(m3, 0.0), 1e-16)
    wsum = (wa + wb) + wc

    idx_out_ref[...] = jnp.concatenate([i1, i2, i3], axis=1)
    wn_out_ref[...] = jnp.concatenate([wa, wb, wc], axis=1) / wsum


_NW = 32
_NSPLIT = 2
_LH = L // _NSPLIT
_QW = _LH // _NW
_CQ = 32
_NCH = _QW // _CQ


def _interp_sc_kernel(f1_hbm, idx_hbm, wn_hbm, f2_hbm, out_hbm,
                      idx_a, idx_b, wn_a, wn_b, f2_a, f2_b,
                      rows_a, rows_b, out_a, out_b,
                      sem_in_a, sem_in_b, sem_g_a, sem_g_b,
                      sem_out_a, sem_out_b):
    wid = lax.axis_index("s") * 2 + lax.axis_index("c")
    idx_v = (idx_a, idx_b)
    wn_v = (wn_a, wn_b)
    f2_v = (f2_a, f2_b)
    rows_v = (rows_a, rows_b)
    out_v = (out_a, out_b)
    sem_in = (sem_in_a, sem_in_b)
    sem_g = (sem_g_a, sem_g_b)
    sem_out = (sem_out_a, sem_out_b)

    def base(c):
        return wid * _QW + c * _CQ

    def fire_in(c):
        b = c & 1
        pltpu.async_copy(idx_hbm.at[pl.ds(base(c) * 3, 3 * _CQ)], idx_v[b], sem_in[b])
        pltpu.async_copy(wn_hbm.at[pl.ds(base(c) * 3, 3 * _CQ)], wn_v[b], sem_in[b])
        pltpu.async_copy(f2_hbm.at[pl.ds(base(c), _CQ)], f2_v[b], sem_in[b])

    def wait_in(c):
        b = c & 1
        pltpu.make_async_copy(idx_hbm.at[pl.ds(base(c) * 3, 3 * _CQ)], idx_v[b], sem_in[b]).wait()
        pltpu.make_async_copy(wn_hbm.at[pl.ds(base(c) * 3, 3 * _CQ)], wn_v[b], sem_in[b]).wait()
        pltpu.make_async_copy(f2_hbm.at[pl.ds(base(c), _CQ)], f2_v[b], sem_in[b]).wait()

    def fire_g(c):
        b = c & 1
        pltpu.async_copy(f1_hbm.at[idx_v[b]], rows_v[b], sem_g[b])

    def wait_g(c):
        b = c & 1
        pltpu.make_async_copy(f1_hbm.at[idx_v[b]], rows_v[b], sem_g[b]).wait()

    def fire_out(c):
        b = c & 1
        pltpu.async_copy(out_v[b], out_hbm.at[pl.ds(base(c), _CQ)], sem_out[b])

    def wait_out(c):
        b = c & 1
        pltpu.make_async_copy(out_v[b], out_hbm.at[pl.ds(base(c), _CQ)], sem_out[b]).wait()

    fire_in(0)
    wait_in(0)
    fire_g(0)
    if _NCH > 1:
        fire_in(1)

    for c in range(_NCH):
        b = c & 1
        if c + 1 < _NCH:
            wait_in(c + 1)
            fire_g(c + 1)
        if c >= 2:
            wait_out(c - 2)
        wait_g(c)
        rows, wns, f2s, outs = rows_v[b], wn_v[b], f2_v[b], out_v[b]

        def qstep(q, _, rows=rows, wns=wns, f2s=f2s, outs=outs):
            w0 = wns[3 * q, :]
            w1 = wns[3 * q + 1, :]
            w2 = wns[3 * q + 2, :]
            for cc in range(OUT // 16):
                sl = pl.ds(cc * 16, 16)
                acc = f2s[q, sl] + w0 * rows[3 * q, sl]
                acc = acc + w1 * rows[3 * q + 1, sl]
                acc = acc + w2 * rows[3 * q + 2, sl]
                outs[q, sl] = acc
            return 0

        lax.fori_loop(0, _CQ, qstep, 0)
        fire_out(c)
        if c + 2 < _NCH:
            fire_in(c + 2)

    wait_out(_NCH - 2)
    wait_out(_NCH - 1)


def _interp_sc(f1, idx_flat, wn_flat, f2):
    return pl.kernel(
        _interp_sc_kernel,
        out_type=jax.ShapeDtypeStruct((_LH, OUT), jnp.float32),
        mesh=plsc.VectorSubcoreMesh(core_axis_name="c", subcore_axis_name="s"),
        scratch_types=[
            pltpu.VMEM((3 * _CQ,), jnp.int32),
            pltpu.VMEM((3 * _CQ,), jnp.int32),
            pltpu.VMEM((3 * _CQ, 16), jnp.float32),
            pltpu.VMEM((3 * _CQ, 16), jnp.float32),
            pltpu.VMEM((_CQ, OUT), jnp.float32),
            pltpu.VMEM((_CQ, OUT), jnp.float32),
            pltpu.VMEM((3 * _CQ, OUT), jnp.float32),
            pltpu.VMEM((3 * _CQ, OUT), jnp.float32),
            pltpu.VMEM((_CQ, OUT), jnp.float32),
            pltpu.VMEM((_CQ, OUT), jnp.float32),
            pltpu.SemaphoreType.DMA,
            pltpu.SemaphoreType.DMA,
            pltpu.SemaphoreType.DMA,
            pltpu.SemaphoreType.DMA,
            pltpu.SemaphoreType.DMA,
            pltpu.SemaphoreType.DMA,
        ],
    )(f1, idx_flat, wn_flat, f2)


def kernel(features_1, positions_1, batch_1, features_2, positions_2, batch_2,
           W1, b1, g1, be1, W2, b2, g2, be2):
    f1 = _branch(features_1, W1, b1, g1, be1)
    f2 = _branch(features_2, W2, b2, g2, be2)

    pos1t = jnp.zeros((8, N), jnp.float32).at[:3].set(positions_1.T)
    b1f = batch_1.astype(jnp.float32).reshape(1, N)
    pos2p = jnp.zeros((L, 8), jnp.float32).at[:, :3].set(positions_2)
    b2f = batch_2.astype(jnp.float32).reshape(L, 1)

    def search(pos2_half, b2_half):
        return pl.pallas_call(
            _search_body,
            grid=(_LH // TQ,),
            in_specs=[
                pl.BlockSpec((TQ, 8), lambda i: (i, 0)),
                pl.BlockSpec((TQ, 1), lambda i: (i, 0)),
                pl.BlockSpec((8, N), lambda i: (0, 0)),
                pl.BlockSpec((1, N), lambda i: (0, 0)),
            ],
            out_specs=[
                pl.BlockSpec((TQ, 3), lambda i: (i, 0)),
                pl.BlockSpec((TQ, 3), lambda i: (i, 0)),
            ],
            out_shape=[
                jax.ShapeDtypeStruct((_LH, 3), jnp.int32),
                jax.ShapeDtypeStruct((_LH, 3), jnp.float32),
            ],
        )(pos2_half, b2_half, pos1t, b1f)

    halves = []
    for h in range(_NSPLIT):
        sl = slice(h * _LH, (h + 1) * _LH)
        idx, wn = search(pos2p[sl], b2f[sl])
        wsplat = jnp.broadcast_to(wn.reshape(3 * _LH)[:, None], (3 * _LH, 16))
        halves.append(_interp_sc(f1, idx.reshape(3 * _LH), wsplat, f2[sl]))

    out = jnp.concatenate(halves, axis=0)
    return (out, positions_2, batch_2)
